# Initial kernel scaffold; baseline (speedup 1.0000x reference)
import jax
import jax.numpy as jnp
from jax.experimental import pallas as pl


def kernel(input_data, codebook, entries):
    raise NotImplementedError("write your pallas kernel here")


# speedup vs baseline: 6.2141x; 6.2141x over previous
import functools

import jax
import jax.numpy as jnp
from jax import lax
from jax.experimental import pallas as pl
from jax.experimental.pallas import tpu as pltpu
from jax.experimental.pallas import tpu_sc as plsc

N = 4096
D = 32
K = 1024
KM1 = K - 1
NC, NS, L = 2, 16, 16
NW = NC * NS
ROWS_PER_W = N // NW
GROUPS = ROWS_PER_W // L
KGROUPS = K // L
BN = 512
NBLK = N // BN

_SC_MESH = plsc.VectorSubcoreMesh(core_axis_name="c", subcore_axis_name="s")


def _wid():
    return lax.axis_index("c") * NS + lax.axis_index("s")


@functools.partial(
    pl.kernel,
    out_type=jax.ShapeDtypeStruct((D, K), jnp.float32),
    mesh=_SC_MESH,
    scratch_types=[
        pltpu.VMEM((1, K), jnp.float32),
        pltpu.VMEM((K,), jnp.int32),
        pltpu.VMEM((K,), jnp.float32),
        pltpu.VMEM((K,), jnp.float32),
        pltpu.SemaphoreType.DMA,
    ],
    compiler_params=pltpu.CompilerParams(needs_layout_passes=False),
)
def _sc_build_dithered(cbT_hbm, i0_hbm, rem_hbm, out_hbm, cb_v, i0_v, rem_v,
                       o_v, sem):
    w = _wid()
    c1h = pltpu.async_copy(cbT_hbm.at[pl.ds(w, 1)], cb_v, sem)
    c2h = pltpu.async_copy(i0_hbm, i0_v, sem)
    c3h = pltpu.async_copy(rem_hbm, rem_v, sem)
    c1h.wait()
    c2h.wait()
    c3h.wait()
    zz = jnp.zeros((L,), jnp.int32)
    for g in range(KGROUPS):
        sl = pl.ds(g * L, L)
        idx = i0_v[sl]
        r = rem_v[sl]
        c0 = plsc.load_gather(cb_v, [zz, idx])
        c1 = plsc.load_gather(cb_v, [zz, idx + 1])
        o_v[sl] = (1.0 - r) * c0 + r * c1
    pltpu.sync_copy(o_v, out_hbm.at[w])


def _tc_top2_body(x_ref, ct_ref, i1_ref, i2_ref):
    x = x_ref[...]
    ct = ct_ref[...]
    p = jnp.dot(x, ct, preferred_element_type=jnp.float32)
    cn = jnp.sum(ct * ct, axis=0, keepdims=True)
    g = cn - 2.0 * p
    iota = lax.broadcasted_iota(jnp.int32, (BN, K), 1)
    big_f = jnp.float32(3e38)
    big_i = jnp.int32(2**30)
    g = jnp.where(iota >= KM1, big_f, g)
    m1 = jnp.min(g, axis=1, keepdims=True)
    i1 = jnp.min(jnp.where(g == m1, iota, big_i), axis=1)
    g2 = jnp.where(iota == i1[:, None], big_f, g)
    m2 = jnp.min(g2, axis=1, keepdims=True)
    i2 = jnp.min(jnp.where(g2 == m2, iota, big_i), axis=1)
    i1_ref[0, 0, :] = i1
    i2_ref[0, 0, :] = i2


_tc_top2 = pl.pallas_call(
    _tc_top2_body,
    grid=(NBLK,),
    in_specs=[
        pl.BlockSpec((BN, D), lambda i: (i, 0)),
        pl.BlockSpec((D, K), lambda i: (0, 0)),
    ],
    out_specs=[
        pl.BlockSpec((1, 1, BN), lambda i: (i, 0, 0)),
        pl.BlockSpec((1, 1, BN), lambda i: (i, 0, 0)),
    ],
    out_shape=[
        jax.ShapeDtypeStruct((NBLK, 1, BN), jnp.int32),
        jax.ShapeDtypeStruct((NBLK, 1, BN), jnp.int32),
    ],
)


@functools.partial(
    pl.kernel,
    out_type=(
        jax.ShapeDtypeStruct((N, D), jnp.float32),
        jax.ShapeDtypeStruct((N,), jnp.int32),
        jax.ShapeDtypeStruct((NW, L, K), jnp.float32),
    ),
    mesh=_SC_MESH,
    scratch_types=[
        pltpu.VMEM((D, K), jnp.float32),
        pltpu.VMEM((ROWS_PER_W, D), jnp.float32),
        pltpu.VMEM((ROWS_PER_W,), jnp.int32),
        pltpu.VMEM((ROWS_PER_W,), jnp.int32),
        pltpu.VMEM((ROWS_PER_W, D), jnp.float32),
        pltpu.VMEM((ROWS_PER_W,), jnp.int32),
        pltpu.VMEM((L, K), jnp.float32),
        pltpu.SemaphoreType.DMA,
    ],
    compiler_params=pltpu.CompilerParams(needs_layout_passes=False),
)
def _sc_rerank(x_hbm, ct_hbm, i1_hbm, i2_hbm, zeros_hbm,
               q_hbm, wi_hbm, hist_hbm,
               ct_v, x_v, i1_v, i2_v, q_v, wi_v, hl_v, sem):
    w = _wid()
    base = w * ROWS_PER_W
    handles = [
        pltpu.async_copy(ct_hbm, ct_v, sem),
        pltpu.async_copy(x_hbm.at[pl.ds(base, ROWS_PER_W)], x_v, sem),
        pltpu.async_copy(i1_hbm.at[pl.ds(base, ROWS_PER_W)], i1_v, sem),
        pltpu.async_copy(i2_hbm.at[pl.ds(base, ROWS_PER_W)], i2_v, sem),
        pltpu.async_copy(zeros_hbm, hl_v, sem),
    ]
    for h in handles:
        h.wait()
    lanes = lax.iota(jnp.int32, L)
    ones = jnp.ones((L,), jnp.float32)
    for g in range(GROUPS):
        sl = pl.ds(g * L, L)
        rowsg = lanes + (g * L)
        i1g = i1_v[sl]
        i2g = i2_v[sl]
        acc1 = jnp.zeros((L,), jnp.float32)
        acc2 = jnp.zeros((L,), jnp.float32)
        for d in range(D):
            dd = jnp.full((L,), d, jnp.int32)
            xd = plsc.load_gather(x_v, [rowsg, dd])
            c1 = plsc.load_gather(ct_v, [dd, i1g])
            c2 = plsc.load_gather(ct_v, [dd, i2g])
            t1 = xd - c1
            acc1 = acc1 + t1 * t1
            t2 = xd - c2
            acc2 = acc2 + t2 * t2
        take1 = (acc1 < acc2) | ((acc1 == acc2) & (i1g < i2g))
        wig = jnp.where(take1, i1g, i2g)
        wi_v[sl] = wig
        plsc.addupdate_scatter(hl_v, [lanes, wig], ones)
        for d in range(D):
            dd = jnp.full((L,), d, jnp.int32)
            qd = plsc.load_gather(ct_v, [dd, wig])
            plsc.store_scatter(q_v, [rowsg, dd], qd)
    oh = [
        pltpu.async_copy(q_v, q_hbm.at[pl.ds(base, ROWS_PER_W)], sem),
        pltpu.async_copy(wi_v, wi_hbm.at[pl.ds(base, ROWS_PER_W)], sem),
        pltpu.async_copy(hl_v, hist_hbm.at[w], sem),
    ]
    for h in oh:
        h.wait()


def _tc_perp_body(h_ref, out_ref):
    h = h_ref[...]
    avg = jnp.sum(h, axis=0, keepdims=True) * (1.0 / N)
    ent = jnp.sum(avg * jnp.log(avg + 1e-10))
    out_ref[0, 0] = jnp.exp(-ent)


_tc_perp = pl.pallas_call(
    _tc_perp_body,
    out_specs=pl.BlockSpec(memory_space=pltpu.SMEM),
    out_shape=jax.ShapeDtypeStruct((1, 1), jnp.float32),
)


def kernel(input_data, codebook, entries):
    dither = jax.random.uniform(jax.random.key(1), (KM1,), dtype=jnp.float32)
    f = dither + jnp.linspace(0.0, float(K - 2), KM1, dtype=jnp.float32)
    f = f + (jnp.asarray(entries) - K).astype(jnp.float32)
    i0 = jnp.clip(jnp.floor(f), 0, K - 2).astype(jnp.int32)
    rem = f - i0.astype(jnp.float32)
    i0p = jnp.concatenate([i0, jnp.zeros((1,), jnp.int32)])
    remp = jnp.concatenate([rem, jnp.zeros((1,), jnp.float32)])
    cbT = codebook.T

    ct = _sc_build_dithered(cbT, i0p, remp)
    i1, i2 = _tc_top2(input_data, ct)
    zeros = jnp.zeros((L, K), jnp.float32)
    q, wi, hist = _sc_rerank(input_data, ct, i1.reshape(N), i2.reshape(N),
                             zeros)
    pp = _tc_perp(hist.reshape(NW * L, K))
    return q, pp.reshape(()), wi
